# baseline (speedup 1.0000x reference)
import functools
import jax
import jax.numpy as jnp
from jax import lax
from jax.experimental import pallas as pl
from jax.experimental.pallas import tpu as pltpu
from jax.experimental.pallas import tpu_sc as plsc

KNN = 40
N = 1024
B = 4
BN = B * N
INT_MIN = -2147483648


def _psum(m):
    lane = lax.broadcasted_iota(jnp.int32, m.shape, 1)
    ps = m
    sh = 1
    while sh < m.shape[1]:
        ps = ps + jnp.where(lane >= sh, jnp.roll(ps, sh, axis=1), 0)
        sh *= 2
    return ps


def _keys_kernel(x_ref, sel_ref):
    xb = x_ref[0]
    inner = jnp.dot(xb, xb.T, preferred_element_type=jnp.float32)
    sq = jnp.sum(xb * xb, axis=1)
    d = 2.0 * inner - sq[:, None] - sq[None, :]
    s = lax.bitcast_convert_type(d, jnp.int32)
    key = jnp.where(s < 0, jnp.int32(INT_MIN) - s, s)

    ones1 = jnp.ones((N, 1), jnp.float32)

    def rowsum(maskf):
        return jnp.dot(maskf, ones1, preferred_element_type=jnp.float32)

    def count_ge(mid):
        return rowsum(jnp.where(key >= mid, 1.0, 0.0))

    c0 = count_ge(jnp.zeros((N, 1), jnp.int32))
    ge40 = c0 >= KNN
    lo = jnp.where(ge40, jnp.int32(0), jnp.int32(INT_MIN))
    hi = jnp.where(ge40, jnp.int32(2147483647), jnp.int32(-1))

    def body(_, carry):
        lo, hi = carry
        w = hi - lo
        mid = lo + (w >> 1) + (w & 1)
        go = count_ge(mid) >= KNN
        return jnp.where(go, mid, lo), jnp.where(go, hi, mid - 1)

    lo, hi = lax.fori_loop(0, 31, body, (lo, hi))
    t = lo

    mgt = key > t
    meq = key == t
    cnt_gt = rowsum(jnp.where(mgt, 1.0, 0.0)).astype(jnp.int32)
    need = KNN - cnt_gt
    eqrank = _psum(jnp.where(meq, 1, 0))
    mask = mgt | (meq & (eqrank <= need))
    pos = _psum(jnp.where(mask, 1, 0))

    posf = pos.astype(jnp.float32)
    cols = []
    for k in range(KNN):
        cols.append(rowsum(jnp.where(posf <= float(k), 1.0, 0.0)))
    sel = jnp.concatenate(cols, axis=1)
    sel_ref[0] = sel.astype(jnp.int32)


def _stage_a(x3d):
    Bq, Nq, C = x3d.shape
    return pl.pallas_call(
        _keys_kernel,
        grid=(Bq,),
        in_specs=[pl.BlockSpec((1, Nq, C), lambda b: (b, 0, 0))],
        out_specs=pl.BlockSpec((1, Nq, KNN), lambda b: (b, 0, 0)),
        out_shape=jax.ShapeDtypeStruct((Bq, Nq, KNN), jnp.int32),
    )(x3d)


RPW = BN // 32
CH = 16
NIDX = (CH * KNN) // 128


NQ = RPW * KNN // 128


def _make_stage_b(Cp):
    mesh = plsc.VectorSubcoreMesh(core_axis_name="c", subcore_axis_name="s")

    @functools.partial(
        pl.kernel,
        mesh=mesh,
        out_type=jax.ShapeDtypeStruct((BN * KNN, Cp), jnp.float32),
        scratch_types=[
            pltpu.VMEM((NQ, 128), jnp.int32),
            pltpu.VMEM((2, 128, Cp), jnp.float32),
            pltpu.SemaphoreType.DMA,
            pltpu.SemaphoreType.DMA,
        ],
    )
    def sck(idx_hbm, x_hbm, xj_out, idxv, buf, sem0, sem1):
        wid = lax.axis_index("s") * 2 + lax.axis_index("c")
        base = wid * (RPW * KNN)
        pltpu.sync_copy(idx_hbm.at[wid], idxv)
        sems = (sem0, sem1)

        def q_iter(gi, carry):
            q0 = gi * 2
            q1 = q0 + 1
            cp0 = pltpu.async_copy(x_hbm.at[idxv.at[q0]], buf.at[0], sem0)
            cp1 = pltpu.async_copy(x_hbm.at[idxv.at[q1]], buf.at[1], sem1)
            cp0.wait()
            pltpu.sync_copy(buf.at[0], xj_out.at[pl.ds(base + q0 * 128, 128)])
            cp1.wait()
            pltpu.sync_copy(buf.at[1], xj_out.at[pl.ds(base + q1 * 128, 128)])
            return carry

        lax.fori_loop(0, NQ // 2, q_iter, 0)

    return sck


def _stage_b(sel, xf):
    boff = (jnp.arange(B, dtype=jnp.int32) * N)[:, None, None]
    idx3 = (sel + boff).reshape(32, NQ, 128)
    return _make_stage_b(xf.shape[-1])(idx3, xf)


def _edge_kernel(xj_ref, x_ref, w_ref, b_ref, o_ref, *, R, Cp, D, Dp):
    xi = x_ref[...]
    xj = xj_ref[...]
    xib = jnp.broadcast_to(xi[:, None, :], (R, KNN, Cp)).reshape(R * KNN, Cp)
    f = jnp.concatenate([xj - xib, xib], axis=1)
    h = jnp.dot(f, w_ref[...].T, preferred_element_type=jnp.float32)
    hm = jnp.max(h.reshape(R, KNN, D), axis=1) + b_ref[...][None, :]
    act = jnp.where(hm >= 0, hm, 0.2 * hm)
    if Dp > D:
        act = jnp.concatenate([act, jnp.zeros((R, Dp - D), jnp.float32)],
                              axis=1)
    o_ref[...] = act


def _stage_c(xj, xf, Wp, b, Dp):
    Cp = xf.shape[-1]
    D = Wp.shape[0]
    R = 128
    nblk = BN // R
    return pl.pallas_call(
        functools.partial(_edge_kernel, R=R, Cp=Cp, D=D, Dp=Dp),
        grid=(nblk,),
        in_specs=[
            pl.BlockSpec((R * KNN, Cp), lambda i: (i, 0)),
            pl.BlockSpec((R, Cp), lambda i: (i, 0)),
            pl.BlockSpec((D, 2 * Cp), lambda i: (0, 0)),
            pl.BlockSpec((D,), lambda i: (0,)),
        ],
        out_specs=pl.BlockSpec((R, Dp), lambda i: (i, 0)),
        out_shape=jax.ShapeDtypeStruct((BN, Dp), jnp.float32),
    )(xj, xf, Wp, b)


def _final_kernel(x1_ref, x2_ref, x3_ref, x4_ref, wf_ref, bf_ref, o_ref):
    outs = []
    for b in range(B):
        sl = pl.ds(b * N, N)
        xc = jnp.concatenate([x1_ref[sl, :], x2_ref[sl, :], x3_ref[sl, :],
                              x4_ref[sl, :]], axis=1)
        h = jnp.dot(xc, wf_ref[...].T, preferred_element_type=jnp.float32)
        outs.append(jnp.max(h, axis=0)[None, :])
    o_ref[...] = jnp.concatenate(outs, axis=0) + bf_ref[...][None, :]


def _stage_d(x1, x2, x3, x4, Wfp, bf):
    return pl.pallas_call(
        _final_kernel,
        out_shape=jax.ShapeDtypeStruct((B, Wfp.shape[0]), jnp.float32),
    )(x1, x2, x3, x4, Wfp, bf)


def _prep_w(W, C, Cp):
    D = W.shape[0]
    if C == Cp:
        return W
    Wp = jnp.zeros((D, 2 * Cp), W.dtype)
    Wp = Wp.at[:, :C].set(W[:, :C])
    Wp = Wp.at[:, Cp:Cp + C].set(W[:, C:])
    return Wp


def kernel(x, W1, b1, W2, b2, W3, b3, W4, b4, Wf, bf):
    x128 = jnp.concatenate([x, jnp.zeros((B, N, 125), jnp.float32)], axis=2)
    xf = x128.reshape(BN, 128)
    x3d = x128
    xs = []
    layers = [(W1, b1, 3, 128, 128), (W2, b2, 64, 128, 128),
              (W3, b3, 64, 128, 128), (W4, b4, 128, 128, 256)]
    for (W, bb, C, Cp, Dp) in layers:
        sel = _stage_a(x3d)
        xj = _stage_b(sel, xf)
        xf = _stage_c(xj, xf, _prep_w(W, C, Cp), bb, Dp)
        x3d = xf.reshape(B, N, Dp)
        xs.append(xf)
    Wfp = jnp.zeros((Wf.shape[0], 640), jnp.float32)
    Wfp = Wfp.at[:, 0:64].set(Wf[:, 0:64])
    Wfp = Wfp.at[:, 128:192].set(Wf[:, 64:128])
    Wfp = Wfp.at[:, 256:384].set(Wf[:, 128:256])
    Wfp = Wfp.at[:, 384:640].set(Wf[:, 256:512])
    return _stage_d(xs[0], xs[1], xs[2], xs[3], Wfp, bf)


# speedup vs baseline: 1.1878x; 1.1878x over previous
import functools
import jax
import jax.numpy as jnp
from jax import lax
from jax.experimental import pallas as pl
from jax.experimental.pallas import tpu as pltpu
from jax.experimental.pallas import tpu_sc as plsc

KNN = 40
N = 1024
B = 4
BN = B * N
INT_MIN = -2147483648


def _psum(m):
    lane = lax.broadcasted_iota(jnp.int32, m.shape, 1)
    ps = m
    sh = 1
    while sh < m.shape[1]:
        ps = ps + jnp.where(lane >= sh, jnp.roll(ps, sh, axis=1), 0)
        sh *= 2
    return ps


NH = N // 2


def _keys_kernel(q_ref, full_ref, sel_ref):
    xq = q_ref[0]
    xb = full_ref[0]
    inner = jnp.dot(xq, xb.T, preferred_element_type=jnp.float32)
    sqq = jnp.sum(xq * xq, axis=1)
    sq = jnp.sum(xb * xb, axis=1)
    d = 2.0 * inner - sqq[:, None] - sq[None, :]
    s = lax.bitcast_convert_type(d, jnp.int32)
    key = jnp.where(s < 0, jnp.int32(INT_MIN) - s, s)

    def count_ge(mid):
        return jnp.sum(jnp.where(key >= mid, 1.0, 0.0), axis=1, keepdims=True)

    c0 = count_ge(jnp.zeros((NH, 1), jnp.int32))
    ge40 = c0 >= KNN
    lo = jnp.where(ge40, jnp.int32(0), jnp.int32(INT_MIN))
    hi = jnp.where(ge40, jnp.int32(2147483647), jnp.int32(-1))

    def body(_, carry):
        lo, hi = carry
        w = hi - lo
        mid = lo + (w >> 1) + (w & 1)
        go = count_ge(mid) >= KNN
        return jnp.where(go, mid, lo), jnp.where(go, hi, mid - 1)

    lo, hi = lax.fori_loop(0, 31, body, (lo, hi))
    t = lo

    mgt = key > t
    meq = key == t
    cnt_gt = jnp.sum(jnp.where(mgt, 1, 0), axis=1, keepdims=True)
    need = KNN - cnt_gt
    eqrank = _psum(jnp.where(meq, 1, 0))
    mask = mgt | (meq & (eqrank <= need))
    pos = _psum(jnp.where(mask, 1, 0))

    cols = []
    for k in range(KNN):
        cols.append(jnp.sum(jnp.where(pos <= k, 1, 0), axis=1, keepdims=True))
    sel_ref[0] = jnp.concatenate(cols, axis=1)


def _stage_a(x3d, h):
    Bq, Nq, C = x3d.shape
    return pl.pallas_call(
        _keys_kernel,
        grid=(Bq,),
        in_specs=[pl.BlockSpec((1, NH, C), lambda b: (b, h, 0)),
                  pl.BlockSpec((1, Nq, C), lambda b: (b, 0, 0))],
        out_specs=pl.BlockSpec((1, NH, KNN), lambda b: (b, 0, 0)),
        out_shape=jax.ShapeDtypeStruct((Bq, NH, KNN), jnp.int32),
    )(x3d, x3d)


HB = B * NH
RPW = HB // 32
NQ = RPW * KNN // 128


def _make_stage_b(Cp):
    mesh = plsc.VectorSubcoreMesh(core_axis_name="c", subcore_axis_name="s")

    @functools.partial(
        pl.kernel,
        mesh=mesh,
        out_type=jax.ShapeDtypeStruct((HB * KNN, Cp), jnp.float32),
        scratch_types=[
            pltpu.VMEM((NQ, 128), jnp.int32),
            pltpu.VMEM((2, 128, Cp), jnp.float32),
            pltpu.SemaphoreType.DMA,
            pltpu.SemaphoreType.DMA,
        ],
    )
    def sck(idx_hbm, x_hbm, xj_out, idxv, buf, sem0, sem1):
        wid = lax.axis_index("s") * 2 + lax.axis_index("c")
        base = wid * (RPW * KNN)
        pltpu.sync_copy(idx_hbm.at[wid], idxv)
        sems = (sem0, sem1)

        def q_iter(gi, carry):
            q0 = gi * 2
            q1 = q0 + 1
            cp0 = pltpu.async_copy(x_hbm.at[idxv.at[q0]], buf.at[0], sem0)
            cp1 = pltpu.async_copy(x_hbm.at[idxv.at[q1]], buf.at[1], sem1)
            cp0.wait()
            pltpu.sync_copy(buf.at[0], xj_out.at[pl.ds(base + q0 * 128, 128)])
            cp1.wait()
            pltpu.sync_copy(buf.at[1], xj_out.at[pl.ds(base + q1 * 128, 128)])
            return carry

        lax.fori_loop(0, NQ // 2, q_iter, 0)

    return sck


def _stage_b(sel, xf):
    boff = (jnp.arange(B, dtype=jnp.int32) * N)[:, None, None]
    idx3 = (sel + boff).reshape(32, NQ, 128)
    return _make_stage_b(xf.shape[-1])(idx3, xf)


def _edge_kernel(xj_ref, x_ref, w_ref, b_ref, o_ref, *, R, Cp, D, Dp):
    xi = x_ref[...]
    xj = xj_ref[...]
    xib = jnp.broadcast_to(xi[:, None, :], (R, KNN, Cp)).reshape(R * KNN, Cp)
    f = jnp.concatenate([xj - xib, xib], axis=1)
    h = jnp.dot(f, w_ref[...].T, preferred_element_type=jnp.float32)
    hm = jnp.max(h.reshape(R, KNN, D), axis=1) + b_ref[...][None, :]
    act = jnp.where(hm >= 0, hm, 0.2 * hm)
    if Dp > D:
        act = jnp.concatenate([act, jnp.zeros((R, Dp - D), jnp.float32)],
                              axis=1)
    o_ref[...] = act


def _stage_c(xj, xf, Wp, b, Dp, h):
    Cp = xf.shape[-1]
    D = Wp.shape[0]
    R = 128
    nblk = HB // R
    bph = NH // R

    def xi_map(i):
        return ((i // bph) * (N // R) + h * bph + (i % bph), 0)

    return pl.pallas_call(
        functools.partial(_edge_kernel, R=R, Cp=Cp, D=D, Dp=Dp),
        grid=(nblk,),
        in_specs=[
            pl.BlockSpec((R * KNN, Cp), lambda i: (i, 0)),
            pl.BlockSpec((R, Cp), xi_map),
            pl.BlockSpec((D, 2 * Cp), lambda i: (0, 0)),
            pl.BlockSpec((D,), lambda i: (0,)),
        ],
        out_specs=pl.BlockSpec((R, Dp), lambda i: (i, 0)),
        out_shape=jax.ShapeDtypeStruct((HB, Dp), jnp.float32),
    )(xj, xf, Wp, b)


def _final_kernel(x1_ref, x2_ref, x3_ref, x4_ref, wf_ref, bf_ref, o_ref):
    outs = []
    for b in range(B):
        sl = pl.ds(b * N, N)
        xc = jnp.concatenate([x1_ref[sl, :], x2_ref[sl, :], x3_ref[sl, :],
                              x4_ref[sl, :]], axis=1)
        h = jnp.dot(xc, wf_ref[...].T, preferred_element_type=jnp.float32)
        outs.append(jnp.max(h, axis=0)[None, :])
    o_ref[...] = jnp.concatenate(outs, axis=0) + bf_ref[...][None, :]


def _stage_d(x1, x2, x3, x4, Wfp, bf):
    return pl.pallas_call(
        _final_kernel,
        out_shape=jax.ShapeDtypeStruct((B, Wfp.shape[0]), jnp.float32),
    )(x1, x2, x3, x4, Wfp, bf)


def _prep_w(W, C, Cp):
    D = W.shape[0]
    if C == Cp:
        return W
    Wp = jnp.zeros((D, 2 * Cp), W.dtype)
    Wp = Wp.at[:, :C].set(W[:, :C])
    Wp = Wp.at[:, Cp:Cp + C].set(W[:, C:])
    return Wp


def kernel(x, W1, b1, W2, b2, W3, b3, W4, b4, Wf, bf):
    x128 = jnp.concatenate([x, jnp.zeros((B, N, 125), jnp.float32)], axis=2)
    xf = x128.reshape(BN, 128)
    x3d = x128
    xs = []
    layers = [(W1, b1, 3, 128, 128), (W2, b2, 64, 128, 128),
              (W3, b3, 64, 128, 128), (W4, b4, 128, 128, 256)]
    for (W, bb, C, Cp, Dp) in layers:
        Wp = _prep_w(W, C, Cp)
        sel0 = _stage_a(x3d, 0)
        sel1 = _stage_a(x3d, 1)
        xj0 = _stage_b(sel0, xf)
        xj1 = _stage_b(sel1, xf)
        y0 = _stage_c(xj0, xf, Wp, bb, Dp, 0)
        y1 = _stage_c(xj1, xf, Wp, bb, Dp, 1)
        xf = jnp.concatenate([y0.reshape(B, 1, NH, Dp),
                              y1.reshape(B, 1, NH, Dp)], axis=1).reshape(BN, Dp)
        x3d = xf.reshape(B, N, Dp)
        xs.append(xf)
    Wfp = jnp.zeros((Wf.shape[0], 640), jnp.float32)
    Wfp = Wfp.at[:, 0:64].set(Wf[:, 0:64])
    Wfp = Wfp.at[:, 128:192].set(Wf[:, 64:128])
    Wfp = Wfp.at[:, 256:384].set(Wf[:, 128:256])
    Wfp = Wfp.at[:, 384:640].set(Wf[:, 256:512])
    return _stage_d(xs[0], xs[1], xs[2], xs[3], Wfp, bf)
